# baseline (speedup 1.0000x reference)
import functools

import jax
import jax.numpy as jnp
from jax import lax
from jax.experimental import pallas as pl
from jax.experimental.pallas import tpu as pltpu
from jax.experimental.pallas import tpu_sc as plsc

_L = 16


def _sc_dots(center, target, negr, v_emb, u_emb, B, K, D, nw, nc):
    bpw = B // nw
    S = 32
    nsub = bpw // S
    spk = S * K
    gpk = spk // _L
    nidx_rows = spk // 128
    nchunk = D // _L

    mesh = plsc.VectorSubcoreMesh(core_axis_name="c", subcore_axis_name="s")

    @functools.partial(
        pl.kernel,
        mesh=mesh,
        compiler_params=pltpu.CompilerParams(
            needs_layout_passes=False, use_tc_tiling_on_sc=False),
        out_type=[
            jax.ShapeDtypeStruct((B,), jnp.float32),
            jax.ShapeDtypeStruct((B * K,), jnp.float32),
        ],
        scratch_types=[
            pltpu.VMEM((bpw,), jnp.int32),
            pltpu.VMEM((bpw,), jnp.int32),
            pltpu.VMEM((spk,), jnp.int32),
            pltpu.VMEM((spk,), jnp.int32),
            pltpu.VMEM((bpw, D), jnp.float32),
            pltpu.VMEM((bpw, D), jnp.float32),
            pltpu.VMEM((spk, D), jnp.float32),
            pltpu.VMEM((spk, D), jnp.float32),
            pltpu.VMEM((spk, _L), jnp.float32),
            pltpu.VMEM((bpw,), jnp.float32),
            pltpu.VMEM((spk,), jnp.float32),
            pltpu.VMEM((spk,), jnp.float32),
            pltpu.SemaphoreType.DMA,
            pltpu.SemaphoreType.DMA,
            pltpu.SemaphoreType.DMA,
            pltpu.SemaphoreType.DMA,
            pltpu.SemaphoreType.DMA,
            pltpu.SemaphoreType.DMA,
        ],
    )
    def sc_kernel(center_hbm, target_hbm, negr_hbm, vemb_hbm, uemb_hbm,
                  pos_hbm, negout_hbm,
                  cidx, tidx, nidx0, nidx1, vrows, urows, nrows0, nrows1,
                  accb, posd, negd0, negd1,
                  semi, semv, semu, semn0, semn1, semd):
        nidx = (nidx0, nidx1)
        nrows = (nrows0, nrows1)
        negd = (negd0, negd1)
        nsem = (semn0, semn1)
        wid = lax.axis_index("s") * nc + lax.axis_index("c")
        base = wid * bpw
        lane = lax.iota(jnp.int32, _L)

        hc = pltpu.async_copy(center_hbm.at[pl.ds(base, bpw)], cidx, semi)
        ht = pltpu.async_copy(target_hbm.at[pl.ds(base, bpw)], tidx, semi)
        hn_idx = pltpu.async_copy(
            negr_hbm.at[pl.ds(base * K, spk)], nidx[0], semi)
        hc.wait()
        ht.wait()
        hv = pltpu.async_copy(vemb_hbm.at[cidx], vrows, semv)
        hu = pltpu.async_copy(uemb_hbm.at[tidx], urows, semu)
        hn_idx.wait()

        def fire_neg_gathers(slot):
            return [
                pltpu.async_copy(
                    uemb_hbm.at[nidx[slot].at[pl.ds(j * 128, 128)]],
                    nrows[slot].at[pl.ds(j * 128, 128)],
                    nsem[slot])
                for j in range(nidx_rows)
            ]

        hn = [None, None]
        hn[0] = fire_neg_gathers(0)

        def reduce_groups(ngroups, goff, out_ref, ooff):
            def group(g, carry):
                rowi = lane + (goff + g * _L)
                tot = plsc.load_gather(accb, [rowi, jnp.zeros((_L,), jnp.int32)])
                for l in range(1, _L):
                    tot = tot + plsc.load_gather(
                        accb, [rowi, jnp.full((_L,), l, jnp.int32)])
                out_ref[pl.ds(ooff + g * _L, _L)] = tot
                return carry
            lax.fori_loop(0, ngroups, group, 0)

        hv.wait()
        hu.wait()

        def pos_row(r, carry):
            acc = urows[r, pl.ds(0, _L)] * vrows[r, pl.ds(0, _L)]
            for c in range(1, nchunk):
                acc = acc + (urows[r, pl.ds(c * _L, _L)]
                             * vrows[r, pl.ds(c * _L, _L)])
            accb[r, :] = acc
            return carry
        lax.fori_loop(0, bpw, pos_row, 0)
        reduce_groups(bpw // _L, 0, posd, 0)
        pltpu.sync_copy(posd, pos_hbm.at[pl.ds(base, bpw)])

        hd = [None, None]
        for s in range(nsub):
            slot = s % 2
            nxt = (s + 1) % 2
            if s + 1 < nsub:
                pltpu.sync_copy(
                    negr_hbm.at[pl.ds((base + (s + 1) * S) * K, spk)],
                    nidx[nxt])
                hn[nxt] = fire_neg_gathers(nxt)
            for h in hn[slot]:
                h.wait()

            def neg_row(j, carry):
                rr = s * S + j
                vc = [vrows[rr, pl.ds(c * _L, _L)] for c in range(nchunk)]
                for k in range(K):
                    row = j * K + k
                    acc = nrows[slot][row, pl.ds(0, _L)] * vc[0]
                    for c in range(1, nchunk):
                        acc = acc + nrows[slot][row, pl.ds(c * _L, _L)] * vc[c]
                    accb[row, :] = acc
                return carry
            lax.fori_loop(0, S, neg_row, 0)

            if hd[slot] is not None:
                hd[slot].wait()
            reduce_groups(gpk, 0, negd[slot], 0)
            hd[slot] = pltpu.async_copy(
                negd[slot],
                negout_hbm.at[pl.ds((base + s * S) * K, spk)],
                semd)
        for h in hd:
            if h is not None:
                h.wait()

    return sc_kernel(center, target, negr, v_emb, u_emb)


def _tc_loss(pos2d, neg2d, B):
    def body(pos_ref, neg_ref, o_ref):
        lp = jax.nn.log_sigmoid(pos_ref[...])
        ln = jax.nn.log_sigmoid(-neg_ref[...])
        o_ref[0, 0] = -(jnp.sum(lp) + jnp.sum(ln)) / jnp.float32(B)

    out = pl.pallas_call(
        body,
        out_shape=jax.ShapeDtypeStruct((1, 1), jnp.float32),
        out_specs=pl.BlockSpec(memory_space=pltpu.SMEM),
    )(pos2d, neg2d)
    return out[0, 0]


def kernel(center, target, negative, v_emb, u_emb):
    B = center.shape[0]
    K = negative.shape[1]
    D = v_emb.shape[1]

    info = plsc.get_sparse_core_info()
    nc, ns = info.num_cores, info.num_subcores
    nw = nc * ns

    center = center.astype(jnp.int32)
    target = target.astype(jnp.int32)
    negr = negative.astype(jnp.int32).reshape(B * K)

    pos, negdots = _sc_dots(center, target, negr, v_emb, u_emb, B, K, D, nw, nc)

    pos2d = pos.reshape(B // 128, 128)
    neg2d = negdots.reshape(B * K // 128, 128)
    return _tc_loss(pos2d, neg2d, B)
